# baseline (speedup 1.0000x reference)
import functools
import jax
import jax.numpy as jnp
from jax import lax
from jax.experimental import pallas as pl
from jax.experimental.pallas import tpu as pltpu
from jax.experimental.pallas import tpu_sc as plsc

NC = 2
NS = 16
NW = NC * NS


@functools.lru_cache(maxsize=None)
def _gather_kernel(B, D, C):
    n_per_w = B // NW
    n_chunks = n_per_w // C
    mesh = plsc.VectorSubcoreMesh(
        core_axis_name="c", subcore_axis_name="s",
        num_cores=NC, num_subcores=NS)

    @functools.partial(
        pl.kernel,
        out_type=jax.ShapeDtypeStruct((B, D), jnp.float32),
        mesh=mesh,
        scratch_types=[
            pltpu.VMEM((n_chunks, C), jnp.int32),
            pltpu.VMEM((C, D), jnp.float32),
            pltpu.SemaphoreType.DMA,
        ],
        compiler_params=pltpu.CompilerParams(use_tc_tiling_on_sc=False),
    )
    def k(idx_hbm, table_hbm, out_hbm, idx_v, rows_v, sem):
        wid = lax.axis_index("s") * NC + lax.axis_index("c")
        pltpu.sync_copy(idx_hbm.at[wid], idx_v)
        base = wid * n_per_w

        def body(g, carry):
            pltpu.async_copy(table_hbm.at[idx_v.at[g]], rows_v, sem).wait()
            pltpu.sync_copy(rows_v, out_hbm.at[pl.ds(base + g * C, C)])
            return carry

        lax.fori_loop(0, n_chunks, body, 0)

    return k


def kernel(x, table):
    B = x.shape[0] * x.shape[1]
    D = table.shape[1]
    C = 128
    idx = x.reshape(NW, (B // NW) // C, C).astype(jnp.int32)
    out = _gather_kernel(B, D, C)(idx, table)
    return out.reshape(x.shape[0], x.shape[1], D)


# speedup vs baseline: 1.1150x; 1.1150x over previous
import functools
import jax
import jax.numpy as jnp
from jax import lax
from jax.experimental import pallas as pl
from jax.experimental.pallas import tpu as pltpu
from jax.experimental.pallas import tpu_sc as plsc

NC = 2
NS = 16
NW = NC * NS
C = 128
S = 512
K = S // C


@functools.lru_cache(maxsize=None)
def _gather_kernel(B, D):
    n_per_w = B // NW
    n_chunks = n_per_w // C
    n_super = n_per_w // S
    n_pairs = n_super // 2
    mesh = plsc.VectorSubcoreMesh(
        core_axis_name="c", subcore_axis_name="s",
        num_cores=NC, num_subcores=NS)

    @functools.partial(
        pl.kernel,
        out_type=jax.ShapeDtypeStruct((B, D), jnp.float32),
        mesh=mesh,
        scratch_types=[
            pltpu.VMEM((n_chunks, C), jnp.int32),
            pltpu.VMEM((S, D), jnp.float32),
            pltpu.VMEM((S, D), jnp.float32),
            pltpu.SemaphoreType.DMA,
            pltpu.SemaphoreType.DMA,
            pltpu.SemaphoreType.DMA,
            pltpu.SemaphoreType.DMA,
        ],
        compiler_params=pltpu.CompilerParams(use_tc_tiling_on_sc=False),
    )
    def k(idx_hbm, table_hbm, out_hbm, idx_v, buf_a, buf_b, ga, gb, sa, sb):
        wid = lax.axis_index("s") * NC + lax.axis_index("c")
        pltpu.sync_copy(idx_hbm.at[wid], idx_v)
        base = wid * n_per_w

        def fire_gathers(s, buf, sem):
            for j in range(K):
                pltpu.async_copy(table_hbm.at[idx_v.at[s * K + j]],
                                 buf.at[pl.ds(j * C, C)], sem)

        def wait_gathers(buf, sem):
            pltpu.make_async_copy(table_hbm.at[pl.ds(0, S)], buf, sem).wait()

        def fire_store(s, buf, sem):
            pltpu.async_copy(buf, out_hbm.at[pl.ds(base + s * S, S)], sem)

        def wait_store(s, buf, sem):
            pltpu.make_async_copy(
                buf, out_hbm.at[pl.ds(base + s * S, S)], sem).wait()

        fire_gathers(0, buf_a, ga)

        wait_gathers(buf_a, ga)
        fire_store(0, buf_a, sa)
        fire_gathers(1, buf_b, gb)
        wait_gathers(buf_b, gb)
        fire_store(1, buf_b, sb)
        wait_store(0, buf_a, sa)
        fire_gathers(2, buf_a, ga)

        def body(i, carry):
            wait_gathers(buf_a, ga)
            fire_store(2 * i, buf_a, sa)
            wait_store(2 * i - 1, buf_b, sb)
            fire_gathers(2 * i + 1, buf_b, gb)
            wait_gathers(buf_b, gb)
            fire_store(2 * i + 1, buf_b, sb)
            wait_store(2 * i, buf_a, sa)
            fire_gathers(2 * i + 2, buf_a, ga)
            return carry

        lax.fori_loop(1, n_pairs - 1, body, 0)

        a_s = n_super - 2
        wait_gathers(buf_a, ga)
        fire_store(a_s, buf_a, sa)
        wait_store(a_s - 1, buf_b, sb)
        fire_gathers(a_s + 1, buf_b, gb)
        wait_gathers(buf_b, gb)
        fire_store(a_s + 1, buf_b, sb)
        wait_store(a_s, buf_a, sa)
        wait_store(a_s + 1, buf_b, sb)

    return k


def kernel(x, table):
    B = x.shape[0] * x.shape[1]
    D = table.shape[1]
    idx = x.reshape(NW, (B // NW) // C, C).astype(jnp.int32)
    out = _gather_kernel(B, D)(idx, table)
    return out.reshape(x.shape[0], x.shape[1], D)


# speedup vs baseline: 1.3623x; 1.2218x over previous
import functools
import jax
import jax.numpy as jnp
from jax import lax
from jax.experimental import pallas as pl
from jax.experimental.pallas import tpu as pltpu
from jax.experimental.pallas import tpu_sc as plsc

NC = 2
NS = 16
NW = NC * NS
C = 128
S = 256
K = S // C
DP = 128


@functools.lru_cache(maxsize=None)
def _gather_kernel(B):
    n_per_w = B // NW
    n_chunks = n_per_w // C
    n_super = n_per_w // S
    n_pairs = n_super // 2
    mesh = plsc.VectorSubcoreMesh(
        core_axis_name="c", subcore_axis_name="s",
        num_cores=NC, num_subcores=NS)

    @functools.partial(
        pl.kernel,
        out_type=jax.ShapeDtypeStruct((B, DP), jnp.float32),
        mesh=mesh,
        scratch_types=[
            pltpu.VMEM((n_chunks, C), jnp.int32),
            pltpu.VMEM((S, DP), jnp.float32),
            pltpu.VMEM((S, DP), jnp.float32),
            pltpu.SemaphoreType.DMA,
            pltpu.SemaphoreType.DMA,
            pltpu.SemaphoreType.DMA,
            pltpu.SemaphoreType.DMA,
        ],
        compiler_params=pltpu.CompilerParams(use_tc_tiling_on_sc=True),
    )
    def k(idx_hbm, table_hbm, out_hbm, idx_v, buf_a, buf_b, ga, gb, sa, sb):
        wid = lax.axis_index("s") * NC + lax.axis_index("c")
        pltpu.sync_copy(idx_hbm.at[wid], idx_v)
        base = wid * n_per_w

        def fire_gathers(s, buf, sem):
            for j in range(K):
                pltpu.async_copy(table_hbm.at[idx_v.at[s * K + j]],
                                 buf.at[pl.ds(j * C, C)], sem)

        def wait_gathers(buf, sem):
            pltpu.make_async_copy(table_hbm.at[pl.ds(0, S)], buf, sem).wait()

        def fire_store(s, buf, sem):
            pltpu.async_copy(buf, out_hbm.at[pl.ds(base + s * S, S)], sem)

        def wait_store(s, buf, sem):
            pltpu.make_async_copy(
                buf, out_hbm.at[pl.ds(base + s * S, S)], sem).wait()

        fire_gathers(0, buf_a, ga)

        wait_gathers(buf_a, ga)
        fire_store(0, buf_a, sa)
        fire_gathers(1, buf_b, gb)
        wait_gathers(buf_b, gb)
        fire_store(1, buf_b, sb)
        wait_store(0, buf_a, sa)
        fire_gathers(2, buf_a, ga)

        def body(i, carry):
            wait_gathers(buf_a, ga)
            fire_store(2 * i, buf_a, sa)
            wait_store(2 * i - 1, buf_b, sb)
            fire_gathers(2 * i + 1, buf_b, gb)
            wait_gathers(buf_b, gb)
            fire_store(2 * i + 1, buf_b, sb)
            wait_store(2 * i, buf_a, sa)
            fire_gathers(2 * i + 2, buf_a, ga)
            return carry

        lax.fori_loop(1, n_pairs - 1, body, 0)

        a_s = n_super - 2
        wait_gathers(buf_a, ga)
        fire_store(a_s, buf_a, sa)
        wait_store(a_s - 1, buf_b, sb)
        fire_gathers(a_s + 1, buf_b, gb)
        wait_gathers(buf_b, gb)
        fire_store(a_s + 1, buf_b, sb)
        wait_store(a_s, buf_a, sa)
        wait_store(a_s + 1, buf_b, sb)

    return k


def kernel(x, table):
    B = x.shape[0] * x.shape[1]
    idx = x.reshape(NW, (B // NW) // C, C).astype(jnp.int32)
    table_wide = jnp.pad(table, ((0, 0), (0, DP - table.shape[1])))
    out = _gather_kernel(B)(idx, table_wide)
    return out.reshape(x.shape[0], x.shape[1], DP)[:, :, :table.shape[1]]
